# baseline (device time: 743738 ns/iter reference)
import jax
import jax.numpy as jnp
from jax import lax
from jax.experimental import pallas as pl
from jax.experimental.pallas import tpu as pltpu

N = 32


def kernel(x, Win0, Wout0, Win1, Wout1, Win2, Wout2):
    B, D = x.shape
    M = N * B

    def body(x_ref, win0, wout0, win1, wout1, win2, wout2, out_ref,
             xfull, part, rs_buf,
             xag_send, xag_recv, rs_send, rs_recv, ag_send, ag_recv):
        me = lax.axis_index("i")
        right = lax.rem(me + 1, N)
        left = lax.rem(me + N - 1, N)

        barrier = pltpu.get_barrier_semaphore()
        pl.semaphore_signal(barrier, inc=1, device_id=(left,),
                            device_id_type=pl.DeviceIdType.MESH)
        pl.semaphore_signal(barrier, inc=1, device_id=(right,),
                            device_id_type=pl.DeviceIdType.MESH)
        pl.semaphore_wait(barrier, 2)

        def ring_ag(target, off, send_sems, recv_sems):
            for h in range(N - 1):
                cs = lax.rem(me + off + 2 * N - h, N)
                cr = lax.rem(me + off + 2 * N - h - 1, N)
                send = pltpu.make_async_remote_copy(
                    src_ref=target.at[pl.ds(cs * B, B), :],
                    dst_ref=target.at[pl.ds(cs * B, B), :],
                    send_sem=send_sems.at[h], recv_sem=recv_sems.at[h],
                    device_id=(right,), device_id_type=pl.DeviceIdType.MESH)
                send.start()
                send.wait_send()
                recv = pltpu.make_async_remote_copy(
                    src_ref=target.at[pl.ds(cr * B, B), :],
                    dst_ref=target.at[pl.ds(cr * B, B), :],
                    send_sem=send_sems.at[h], recv_sem=recv_sems.at[h],
                    device_id=(right,), device_id_type=pl.DeviceIdType.MESH)
                recv.wait_recv()

        xfull[pl.ds(me * B, B), :] = x_ref[:, :]
        ring_ag(xfull, 0, xag_send, xag_recv)

        layers = ((win0, wout0), (win1, wout1), (win2, wout2))
        for l, (win, wout) in enumerate(layers):
            hmat = jnp.maximum(
                jnp.dot(xfull[:, :], win[:, :],
                        preferred_element_type=jnp.float32), 0.0)
            part[:, :] = jnp.dot(hmat, wout[:, :],
                                 preferred_element_type=jnp.float32)

            for h in range(N - 1):
                cs = lax.rem(me + 2 * N - h, N)
                cr = lax.rem(me + 2 * N - h - 1, N)
                rdma = pltpu.make_async_remote_copy(
                    src_ref=part.at[pl.ds(cs * B, B), :],
                    dst_ref=rs_buf.at[h],
                    send_sem=rs_send.at[h], recv_sem=rs_recv.at[h],
                    device_id=(right,), device_id_type=pl.DeviceIdType.MESH)
                rdma.start()
                rdma.wait()
                part[pl.ds(cr * B, B), :] = (
                    part[pl.ds(cr * B, B), :] + rs_buf[h, :, :])

            target = out_ref if l == 2 else xfull
            c_red = lax.rem(me + 1, N)
            target[pl.ds(c_red * B, B), :] = part[pl.ds(c_red * B, B), :]
            ring_ag(target, 1, ag_send, ag_recv)

    return pl.pallas_call(
        body,
        out_shape=jax.ShapeDtypeStruct((M, D), jnp.float32),
        in_specs=[pl.BlockSpec(memory_space=pltpu.VMEM)] * 7,
        out_specs=pl.BlockSpec(memory_space=pltpu.VMEM),
        scratch_shapes=[
            pltpu.VMEM((M, D), jnp.float32),
            pltpu.VMEM((M, D), jnp.float32),
            pltpu.VMEM((N - 1, B, D), jnp.float32),
            pltpu.SemaphoreType.DMA((N - 1,)),
            pltpu.SemaphoreType.DMA((N - 1,)),
            pltpu.SemaphoreType.DMA((N - 1,)),
            pltpu.SemaphoreType.DMA((N - 1,)),
            pltpu.SemaphoreType.DMA((N - 1,)),
            pltpu.SemaphoreType.DMA((N - 1,)),
        ],
        compiler_params=pltpu.CompilerParams(collective_id=0),
    )(x, Win0, Wout0, Win1, Wout1, Win2, Wout2)
